# Initial kernel scaffold; baseline (speedup 1.0000x reference)
import jax
import jax.numpy as jnp
from jax.experimental import pallas as pl


def kernel(x, edge_index, edge_attr, A_edge_index, A_edge_weight, d, hop_bias, Wk, bk, Wq, bq, Wv, bv, Wskip, bias):
    raise NotImplementedError("write your pallas kernel here")


# speedup vs baseline: 7.9878x; 7.9878x over previous
import functools

import jax
import jax.numpy as jnp
from jax import lax
from jax.experimental import pallas as pl
from jax.experimental.pallas import tpu as pltpu
from jax.experimental.pallas import tpu_sc as plsc

_N = 10000
_E = 320000
_C = 128
_P = 3

_NC = 2
_NS = 16
_NW = _NC * _NS
_EPT = _E // _NW
_G = 80
_SB = 5
_NSU = _EPT // (_SB * _G)
_RPT = 640
_DW = 16
_L = 16
_NP = 10240
_DRW = _NP // 128


def _sc_mesh():
    return plsc.VectorSubcoreMesh(
        core_axis_name="c", subcore_axis_name="s",
        num_cores=_NC, num_subcores=_NS)


@functools.partial(
    pl.kernel,
    out_type=jax.ShapeDtypeStruct((_NC, _P, _DRW, 128), jnp.float32),
    mesh=_sc_mesh(),
    compiler_params=pltpu.CompilerParams(needs_layout_passes=False),
    scratch_types=[
        pltpu.VMEM((_NSU * _SB, _G), jnp.int32),
        pltpu.VMEM((_DRW, 128), jnp.float32),
        pltpu.VMEM((1, _DRW), jnp.int32),
        pltpu.VMEM_SHARED((_DRW, 128), jnp.float32),
    ],
)
def _deg_kernel(ei_hbm, out_hbm, colbuf, degbuf, ident, shared_deg):
    cid = lax.axis_index("c")
    sid = lax.axis_index("s")
    wid = cid * _NS + sid
    ones16 = jnp.ones((_L,), jnp.float32)

    for i in range(_DRW // _L):
        ident[0, pl.ds(i * _L, _L)] = lax.iota(jnp.int32, _L) + i * _L

    for p in range(_P):
        def zero(i, _):
            for j in range(128 // _L):
                degbuf[i, pl.ds(j * _L, _L)] = jnp.zeros((_L,), jnp.float32)
            return 0
        lax.fori_loop(0, _DRW, zero, 0)
        @pl.when(sid == 0)
        def _():
            pltpu.sync_copy(degbuf, shared_deg)
        plsc.subcore_barrier()

        pltpu.sync_copy(ei_hbm.at[p, 1, wid], colbuf)

        def scat(ch, _):
            for i in range(_G // _L):
                c16 = colbuf[ch, pl.ds(i * _L, _L)]
                plsc.addupdate_scatter(
                    degbuf, [jnp.right_shift(c16, 7),
                             jnp.bitwise_and(c16, 127)], ones16)
            return 0
        lax.fori_loop(0, _NSU * _SB, scat, 0)

        pltpu.sync_copy(degbuf, shared_deg.at[ident.at[0]], add=True)
        plsc.subcore_barrier()

        @pl.when(sid == 0)
        def _(p=p):
            pltpu.sync_copy(shared_deg, out_hbm.at[cid, p])
        plsc.subcore_barrier()


def _dis_body(dp_ref, d0_ref, d1_ref, d2_ref):
    deg = dp_ref[0] + dp_ref[1]
    safe = jnp.where(deg > 0.5, deg, 1.0)
    dis = jnp.where(deg > 0.5, lax.rsqrt(safe), 0.0)
    d0_ref[...] = dis[0]
    d1_ref[...] = dis[1]
    d2_ref[...] = dis[2]


_dis_call = pl.pallas_call(
    _dis_body,
    out_shape=[jax.ShapeDtypeStruct((_DRW, 128), jnp.float32)] * _P,
)


_RB = 1000


def _dense_body(x_ref, d_ref, hb_ref, Wk_ref, bk_ref, Wq_ref, bq_ref,
                Wv_ref, bv_ref, Ws_ref, bias_ref,
                kk_ref, qq_ref, vv_ref, skip_ref):
    xb = x_ref[...]
    dmat = d_ref[...]
    m = jnp.max(dmat, axis=0, keepdims=True)
    e = jnp.exp(dmat - m)
    dw = e / jnp.sum(e, axis=0, keepdims=True)

    dn = (((1,), (1,)), ((), ()))
    for p in range(_P):
        kk_ref[p] = lax.dot_general(
            xb, Wk_ref[p], dn, preferred_element_type=jnp.float32
        ) + bk_ref[p][None, :]
        qq_ref[p] = lax.dot_general(
            xb, Wq_ref[p], dn, preferred_element_type=jnp.float32
        ) + bq_ref[p][None, :]
        vv_ref[p] = (lax.dot_general(
            xb, Wv_ref[p], dn, preferred_element_type=jnp.float32
        ) + bv_ref[p][None, :]) * dw[p][None, :]

    Wc = jnp.sum(dw[:, :, None] * Ws_ref[...], axis=0)
    bc = jnp.sum(dw * bias_ref[...], axis=0) + hb_ref[...]
    skip_ref[...] = lax.dot_general(
        xb, Wc, dn, preferred_element_type=jnp.float32) + bc[None, :]


_dense_call = pl.pallas_call(
    _dense_body,
    grid=(_N // _RB,),
    in_specs=[
        pl.BlockSpec((_RB, _C), lambda i: (i, 0)),
        pl.BlockSpec((_P, _C), lambda i: (0, 0)),
        pl.BlockSpec((_C,), lambda i: (0,)),
        pl.BlockSpec((_P, _C, _C), lambda i: (0, 0, 0)),
        pl.BlockSpec((_P, _C), lambda i: (0, 0)),
        pl.BlockSpec((_P, _C, _C), lambda i: (0, 0, 0)),
        pl.BlockSpec((_P, _C), lambda i: (0, 0)),
        pl.BlockSpec((_P, _C, _C), lambda i: (0, 0, 0)),
        pl.BlockSpec((_P, _C), lambda i: (0, 0)),
        pl.BlockSpec((_P, _C, _C), lambda i: (0, 0, 0)),
        pl.BlockSpec((_P, _C), lambda i: (0, 0)),
    ],
    out_specs=[
        pl.BlockSpec((_P, _RB, _C), lambda i: (0, i, 0)),
        pl.BlockSpec((_P, _RB, _C), lambda i: (0, i, 0)),
        pl.BlockSpec((_P, _RB, _C), lambda i: (0, i, 0)),
        pl.BlockSpec((_RB, _C), lambda i: (i, 0)),
    ],
    out_shape=[
        jax.ShapeDtypeStruct((_P, _N, _C), jnp.float32),
        jax.ShapeDtypeStruct((_P, _N, _C), jnp.float32),
        jax.ShapeDtypeStruct((_P, _N, _C), jnp.float32),
        jax.ShapeDtypeStruct((_N, _C), jnp.float32),
    ],
)


@functools.partial(
    pl.kernel,
    out_type=jax.ShapeDtypeStruct((_NC, _N, _C), jnp.float32),
    mesh=_sc_mesh(),
    compiler_params=pltpu.CompilerParams(needs_layout_passes=False),
    scratch_types=[
        pltpu.VMEM((_DRW, 128), jnp.float32),
        pltpu.VMEM((_SB, _G), jnp.int32),
        pltpu.VMEM((_SB, _G), jnp.int32),
        pltpu.VMEM((_SB, _G), jnp.float32),
        pltpu.VMEM((_SB, _G), jnp.float32),
        pltpu.VMEM((_G, _C), jnp.float32),
        pltpu.VMEM((_G, _C), jnp.float32),
        pltpu.VMEM((_G, _C), jnp.float32),
        pltpu.VMEM((_G,), jnp.float32),
        pltpu.SemaphoreType.DMA,
        pltpu.SemaphoreType.DMA,
        pltpu.SemaphoreType.DMA,
        pltpu.VMEM_SHARED((_N, _C), jnp.float32),
    ],
)
def _edge_kernel(kk, qq, vv, d0, d1, d2, ei, ew, ea, out_hbm,
                 dis_v, rbuf, cbuf, wbuf, abuf, kbuf, qbuf, vbuf, nbuf,
                 semk, semq, semv, shared_agg):
    cid = lax.axis_index("c")
    sid = lax.axis_index("s")
    wid = cid * _NS + sid

    def fill_zero(i, _):
        for j in range(_C // _L):
            kbuf[i, pl.ds(j * _L, _L)] = jnp.zeros((_L,), jnp.float32)
        return 0
    lax.fori_loop(0, _G, fill_zero, 0)
    stripe = pl.multiple_of(jnp.minimum(sid * _RPT, _N - _RPT), 8)
    for t in range(_RPT // _G):
        pltpu.sync_copy(kbuf, shared_agg.at[pl.ds(stripe + t * _G, _G)])
    plsc.subcore_barrier()

    dhs = (d0, d1, d2)
    for p in range(_P):
        pltpu.sync_copy(dhs[p], dis_v)

        def super_body(su, _, p=p):
            pltpu.sync_copy(ei.at[p, 0, wid, su], rbuf)
            pltpu.sync_copy(ei.at[p, 1, wid, su], cbuf)
            pltpu.sync_copy(ew.at[p, wid, su], wbuf)
            if p == 0:
                pltpu.sync_copy(ea.at[wid, su], abuf)

            def chunk_body(g, _, p=p):
                dk = pltpu.async_copy(kk.at[p].at[cbuf.at[g]], kbuf, semk)
                dq = pltpu.async_copy(qq.at[p].at[rbuf.at[g]], qbuf, semq)
                dv = pltpu.async_copy(vv.at[p].at[rbuf.at[g]], vbuf, semv)

                for i in range(_G // _L):
                    sl = pl.ds(i * _L, _L)
                    r16 = rbuf[g, sl]
                    c16 = cbuf[g, sl]
                    nrm = (plsc.load_gather(
                               dis_v, [jnp.right_shift(r16, 7),
                                       jnp.bitwise_and(r16, 127)])
                           * plsc.load_gather(
                               dis_v, [jnp.right_shift(c16, 7),
                                       jnp.bitwise_and(c16, 127)])
                           * wbuf[g, sl])
                    if p == 0:
                        nrm = nrm * abuf[g, sl]
                    nbuf[sl] = nrm
                dk.wait()
                dq.wait()
                dv.wait()

                def edge_body(i, _):
                    n16 = nbuf[pl.ds(i * _L, _L)]
                    for l in range(_L):
                        ne = jnp.broadcast_to(n16[l], (_L,))
                        e = i * _L + l
                        for j in range(_C // _L):
                            sl = pl.ds(j * _L, _L)
                            z = kbuf[e, sl] + qbuf[e, sl]
                            sg = 1.0 / (1.0 + jnp.exp(-z))
                            vbuf[e, sl] = ne * sg * vbuf[e, sl]
                    return 0
                lax.fori_loop(0, _G // _L, edge_body, 0)

                pltpu.sync_copy(vbuf, shared_agg.at[cbuf.at[g]], add=True)
                return 0
            lax.fori_loop(0, _SB, chunk_body, 0)
            return 0
        lax.fori_loop(0, _NSU, super_body, 0)

    plsc.subcore_barrier()
    rows = pl.ds(stripe, _RPT)
    pltpu.sync_copy(shared_agg.at[rows], out_hbm.at[cid, rows])


def _combine_body(parts_ref, skip_ref, out_ref):
    out_ref[...] = parts_ref[0] + parts_ref[1] + skip_ref[...]


_combine_call = pl.pallas_call(
    _combine_body,
    grid=(_N // _RB,),
    in_specs=[
        pl.BlockSpec((_NC, _RB, _C), lambda i: (0, i, 0)),
        pl.BlockSpec((_RB, _C), lambda i: (i, 0)),
    ],
    out_specs=pl.BlockSpec((_RB, _C), lambda i: (i, 0)),
    out_shape=jax.ShapeDtypeStruct((_N, _C), jnp.float32),
)


def kernel(x, edge_index, edge_attr, A_edge_index, A_edge_weight, d,
           hop_bias, Wk, bk, Wq, bq, Wv, bv, Wskip, bias):
    del edge_index
    ei32 = A_edge_index.astype(jnp.int32)
    ei = ei32.reshape(_P, 2, _NW, _NSU, _SB, _G)
    ei2 = ei32.reshape(_P, 2, _NW, _NSU * _SB, _G)
    ew = A_edge_weight.reshape(_P, _NW, _NSU, _SB, _G)
    ea = edge_attr.reshape(_NW, _NSU, _SB, _G)

    degparts = _deg_kernel(ei2)
    d0, d1, d2 = _dis_call(degparts)
    kk, qq, vv, skip = _dense_call(x, d, hop_bias, Wk, bk, Wq, bq, Wv, bv,
                                   Wskip, bias)
    parts = _edge_kernel(kk, qq, vv, d0, d1, d2, ei, ew, ea)
    return _combine_call(parts, skip)
